# Initial kernel scaffold; baseline (speedup 1.0000x reference)
import jax
import jax.numpy as jnp
from jax.experimental import pallas as pl


def kernel(z, edge_index, W1_l, b1, W1_r, W2_l, b2, W2_r, W4_l, b4, W4_r):
    raise NotImplementedError("write your pallas kernel here")


# speedup vs baseline: 7.3601x; 7.3601x over previous
import jax
import jax.numpy as jnp
from jax import lax
from jax.experimental import pallas as pl
from jax.experimental.pallas import tpu as pltpu
from jax.experimental.pallas import tpu_sc as plsc

N = 100000
E = 1600000

NSC = 2
NTILE = 16
K = 8
B = K * 128
NB = 49
ET = NB * B
E_PAD = NSC * NTILE * ET
NPAD = 100096
ACC_ROWS = NPAD
JUNK = N
ZROWS = 391
CPR = NPAD // NTILE


def _make_sc_agg(mult, nch):
    def body(table, src2d, dst2d, zeros_hbm, out, idx_v, gidx_v, dst_v,
             rows_v, zeros_v, acc, sem):
        c = lax.axis_index("c")
        s = lax.axis_index("s")
        w = c * NTILE + s
        tile_row0 = w * (ET // 128)
        pltpu.sync_copy(zeros_hbm, zeros_v)

        for k in range(nch):
            for zz in range(16):
                pltpu.sync_copy(
                    zeros_v, acc.at[pl.ds(s * CPR + zz * ZROWS, ZROWS)])
            plsc.subcore_barrier()

            def batch(g, carry):
                row0 = tile_row0 + g * K
                pltpu.sync_copy(src2d.at[pl.ds(row0, K)], idx_v)
                pltpu.sync_copy(dst2d.at[pl.ds(row0, K)], dst_v)
                if mult == 1:
                    gref = idx_v
                else:
                    for j in range(K):
                        for i in range(8):
                            sl = pl.ds(i * 16, 16)
                            gidx_v[j, sl] = idx_v[j, sl] * mult + k
                    gref = gidx_v
                handles = [
                    pltpu.async_copy(table.at[gref.at[j]], rows_v.at[j], sem)
                    for j in range(K)
                ]
                for h in handles:
                    h.wait()
                for j in range(K):
                    pltpu.sync_copy(rows_v.at[j], acc.at[dst_v.at[j]],
                                    add=True)
                return carry

            lax.fori_loop(0, NB, batch, 0)
            plsc.subcore_barrier()
            pltpu.sync_copy(acc.at[pl.ds(s * CPR, CPR)],
                            out.at[c, k, pl.ds(s * CPR, CPR)])
            plsc.subcore_barrier()

    return pl.kernel(
        body,
        out_type=jax.ShapeDtypeStruct((NSC, nch, NPAD, 16), jnp.float32),
        mesh=plsc.VectorSubcoreMesh(core_axis_name="c", subcore_axis_name="s"),
        compiler_params=pltpu.CompilerParams(use_tc_tiling_on_sc=False),
        scratch_types=[
            pltpu.VMEM((K, 128), jnp.int32),
            pltpu.VMEM((K, 128), jnp.int32),
            pltpu.VMEM((K, 128), jnp.int32),
            pltpu.VMEM((K, 128, 16), jnp.float32),
            pltpu.VMEM((ZROWS, 16), jnp.float32),
            pltpu.VMEM_SHARED((ACC_ROWS, 16), jnp.float32),
            pltpu.SemaphoreType.DMA,
        ],
    )


_sc_agg_l1 = _make_sc_agg(1, 1)
_sc_agg_l2 = _make_sc_agg(4, 4)
_sc_agg_l3 = _make_sc_agg(3, 3)

BLK = 1024
GRID = (N + BLK - 1) // BLK


def _tc1_body(ap, zp, wl, wr, h1o, invo):
    a = ap[...]
    A = a[0, 0] + a[1, 0]
    inv = 1.0 / jnp.maximum(A[:, 3:4], 1.0)
    h = (jnp.dot(A * inv, wl[...], preferred_element_type=jnp.float32)
         + jnp.dot(zp[...], wr[...], preferred_element_type=jnp.float32))
    h1o[...] = jnp.maximum(h, 0.0)
    invo[...] = jnp.broadcast_to(inv, (inv.shape[0], 16))


def _tc2_body(ap, inv, h1, w2l, w2r, w4l, b2t, h2o, y3o):
    a = ap[...]
    iv = inv[...][:, 0:1]
    w2lk = w2l[...]
    h = (jnp.dot(h1[...], w2r[...], preferred_element_type=jnp.float32)
         + b2t[...][0:1, :])
    for kk in range(4):
        m = (a[0, kk] + a[1, kk]) * iv
        h = h + jnp.dot(m, w2lk[kk], preferred_element_type=jnp.float32)
    h2 = jnp.maximum(h, 0.0)
    h2o[...] = h2
    y3o[...] = jnp.dot(h2, w4l[...], preferred_element_type=jnp.float32)


def _tc3_body(ap, inv, h2, w4r, b4t, sel, oo):
    a = ap[...]
    iv = inv[...][:, 0:1]
    selk = sel[...]
    o = (jnp.dot(h2[...], w4r[...], preferred_element_type=jnp.float32)
         + b4t[...][0:1, :])
    for kk in range(3):
        m = (a[0, kk] + a[1, kk]) * iv
        o = o + jnp.dot(m, selk[kk], preferred_element_type=jnp.float32)
    oo[...] = o[:, :40]


def _rows(lanes):
    return pl.BlockSpec((BLK, lanes), lambda i: (i, 0))


def _chunked(nch):
    return pl.BlockSpec((NSC, nch, BLK, 16), lambda i: (0, 0, i, 0))


def _full(shape):
    return pl.BlockSpec(shape, lambda i: tuple(0 for _ in shape))


def _jnp_agg(table, src, dst, mult, nch):
    E2 = E // 2
    outs = []
    for c in range(2):
        sl = slice(c * E2, (c + 1) * E2)
        rows = []
        for k in range(nch):
            msgs = table[src[sl] * mult + k]
            rows.append(jax.ops.segment_sum(msgs, dst[sl], num_segments=NPAD))
        outs.append(jnp.stack(rows))
    return jnp.stack(outs)


def kernel(z, edge_index, W1_l, b1, W1_r, W2_l, b2, W2_r, W4_l, b4, W4_r):
    f32 = jnp.float32
    zp = jnp.concatenate(
        [z, jnp.ones((N, 1), f32), jnp.zeros((N, 12), f32)], axis=1)
    W1lp = jnp.zeros((16, 64), f32).at[0:3].set(W1_l.T)
    W1rp = jnp.zeros((16, 64), f32).at[0:3].set(W1_r.T).at[3].set(b1)
    w2l = W2_l.T.reshape(4, 16, 64)
    w2r = W2_r.T
    w4l = jnp.zeros((64, 48), f32).at[:, :40].set(W4_l.T)
    b2t = jnp.tile(b2[None, :], (8, 1))
    w4r = jnp.zeros((64, 48), f32).at[:, :40].set(W4_r.T)
    b4t = jnp.zeros((8, 48), f32).at[:, :40].set(jnp.tile(b4[None, :], (8, 1)))
    sel = jnp.stack([
        jnp.pad(jnp.eye(16, dtype=f32), ((0, 0), (16 * k, 48 - 16 * (k + 1))))
        for k in range(3)
    ])
    src = edge_index[0]
    dst = edge_index[1]
    zeros_hbm = jnp.zeros((ZROWS, 16), f32)
    padn = E_PAD - E
    src2d = jnp.concatenate(
        [src, jnp.zeros((padn,), jnp.int32)]).reshape(E_PAD // 128, 128)
    dst2d = jnp.concatenate(
        [dst, jnp.full((padn,), JUNK, jnp.int32)]).reshape(E_PAD // 128, 128)

    agg1 = _sc_agg_l1(zp, src2d, dst2d, zeros_hbm)
    h1, inv16 = pl.pallas_call(
        _tc1_body,
        grid=(GRID,),
        in_specs=[_chunked(1), _rows(16), _full((16, 64)), _full((16, 64))],
        out_specs=[_rows(64), _rows(16)],
        out_shape=[jax.ShapeDtypeStruct((N, 64), f32),
                   jax.ShapeDtypeStruct((N, 16), f32)],
    )(agg1, zp, W1lp, W1rp)

    agg2 = _sc_agg_l2(h1.reshape(4 * N, 16), src2d, dst2d, zeros_hbm)
    h2, y3 = pl.pallas_call(
        _tc2_body,
        grid=(GRID,),
        in_specs=[_chunked(4), _rows(16), _rows(64), _full((4, 16, 64)),
                  _full((64, 64)), _full((64, 48)), _full((8, 64))],
        out_specs=[_rows(64), _rows(48)],
        out_shape=[jax.ShapeDtypeStruct((N, 64), f32),
                   jax.ShapeDtypeStruct((N, 48), f32)],
    )(agg2, inv16, h1, w2l, w2r, w4l, b2t)

    agg3 = _sc_agg_l3(y3.reshape(3 * N, 16), src2d, dst2d, zeros_hbm)
    out = pl.pallas_call(
        _tc3_body,
        grid=(GRID,),
        in_specs=[_chunked(3), _rows(16), _rows(64), _full((64, 48)),
                  _full((8, 48)), _full((3, 16, 48))],
        out_specs=_rows(40),
        out_shape=jax.ShapeDtypeStruct((N, 40), f32),
    )(agg3, inv16, h2, w4r, b4t, sel)
    return out


# speedup vs baseline: 7.8087x; 1.0610x over previous
import jax
import jax.numpy as jnp
from jax import lax
from jax.experimental import pallas as pl
from jax.experimental.pallas import tpu as pltpu
from jax.experimental.pallas import tpu_sc as plsc

N = 100000
E = 1600000

NSC = 2
NTILE = 16
K = 4
B = K * 128
NB2 = 49
ET = NB2 * 2 * B
E_PAD = NSC * NTILE * ET
NPAD = 100096
ACC_ROWS = NPAD
JUNK = N
ZROWS = 1564
CPR = NPAD // NTILE


def _make_sc_agg(mult, nch):
    def body(table, src2d, dst2d, zeros_hbm, out, idx0, idx1, dst0, dst1,
             gidx0, gidx1, rows0, rows1, acc, semg, sems0, sems1):
        c = lax.axis_index("c")
        s = lax.axis_index("s")
        w = c * NTILE + s
        tile_row0 = w * (ET // 128)

        def compute_gidx(gidx, idx, k):
            if mult == 1:
                return idx
            for j in range(K):
                for i in range(8):
                    sl = pl.ds(i * 16, 16)
                    gidx[j, sl] = idx[j, sl] * mult + k
            return gidx

        for k in range(nch):
            for zz in range(4):
                pltpu.sync_copy(
                    zeros_hbm, acc.at[pl.ds(s * CPR + zz * ZROWS, ZROWS)])
            plsc.subcore_barrier()

            def batch2(t, carry):
                row0 = tile_row0 + t * 2 * K
                @pl.when(t > 0)
                def _():
                    for j in range(K):
                        pltpu.make_async_copy(
                            rows0.at[j], acc.at[dst0.at[j]], sems0).wait()
                pltpu.sync_copy(src2d.at[pl.ds(row0, K)], idx0)
                pltpu.sync_copy(dst2d.at[pl.ds(row0, K)], dst0)
                g0 = compute_gidx(gidx0, idx0, k)
                h0 = [pltpu.async_copy(table.at[g0.at[j]], rows0.at[j], semg)
                      for j in range(K)]
                @pl.when(t > 0)
                def _():
                    for j in range(K):
                        pltpu.make_async_copy(
                            rows1.at[j], acc.at[dst1.at[j]], sems1).wait()
                pltpu.sync_copy(src2d.at[pl.ds(row0 + K, K)], idx1)
                pltpu.sync_copy(dst2d.at[pl.ds(row0 + K, K)], dst1)
                g1 = compute_gidx(gidx1, idx1, k)
                h1 = [pltpu.async_copy(table.at[g1.at[j]], rows1.at[j], semg)
                      for j in range(K)]
                for h in h0:
                    h.wait()
                for j in range(K):
                    pltpu.async_copy(rows0.at[j], acc.at[dst0.at[j]], sems0,
                                     add=True)
                for h in h1:
                    h.wait()
                for j in range(K):
                    pltpu.async_copy(rows1.at[j], acc.at[dst1.at[j]], sems1,
                                     add=True)
                return carry

            lax.fori_loop(0, NB2, batch2, 0)
            for j in range(K):
                pltpu.make_async_copy(
                    rows0.at[j], acc.at[dst0.at[j]], sems0).wait()
            for j in range(K):
                pltpu.make_async_copy(
                    rows1.at[j], acc.at[dst1.at[j]], sems1).wait()
            plsc.subcore_barrier()
            pltpu.sync_copy(acc.at[pl.ds(s * CPR, CPR)],
                            out.at[c, k, pl.ds(s * CPR, CPR)])
            plsc.subcore_barrier()

    return pl.kernel(
        body,
        out_type=jax.ShapeDtypeStruct((NSC, nch, NPAD, 16), jnp.float32),
        mesh=plsc.VectorSubcoreMesh(core_axis_name="c", subcore_axis_name="s"),
        compiler_params=pltpu.CompilerParams(use_tc_tiling_on_sc=False),
        scratch_types=[
            pltpu.VMEM((K, 128), jnp.int32),
            pltpu.VMEM((K, 128), jnp.int32),
            pltpu.VMEM((K, 128), jnp.int32),
            pltpu.VMEM((K, 128), jnp.int32),
            pltpu.VMEM((K, 128), jnp.int32),
            pltpu.VMEM((K, 128), jnp.int32),
            pltpu.VMEM((K, 128, 16), jnp.float32),
            pltpu.VMEM((K, 128, 16), jnp.float32),
            pltpu.VMEM_SHARED((ACC_ROWS, 16), jnp.float32),
            pltpu.SemaphoreType.DMA,
            pltpu.SemaphoreType.DMA,
            pltpu.SemaphoreType.DMA,
        ],
    )


_sc_agg_l1 = _make_sc_agg(1, 1)
_sc_agg_l2 = _make_sc_agg(4, 4)
_sc_agg_l3 = _make_sc_agg(3, 3)

BLK = 1024
GRID = (N + BLK - 1) // BLK


def _tc1_body(ap, zp, wl, wr, h1o, invo):
    a = ap[...]
    A = a[0, 0] + a[1, 0]
    inv = 1.0 / jnp.maximum(A[:, 3:4], 1.0)
    h = (jnp.dot(A * inv, wl[...], preferred_element_type=jnp.float32)
         + jnp.dot(zp[...], wr[...], preferred_element_type=jnp.float32))
    h1o[...] = jnp.maximum(h, 0.0)
    invo[...] = jnp.broadcast_to(inv, (inv.shape[0], 16))


def _tc2_body(ap, inv, h1, w2l, w2r, w4l, b2t, h2o, y3o):
    a = ap[...]
    iv = inv[...][:, 0:1]
    w2lk = w2l[...]
    h = (jnp.dot(h1[...], w2r[...], preferred_element_type=jnp.float32)
         + b2t[...][0:1, :])
    for kk in range(4):
        m = (a[0, kk] + a[1, kk]) * iv
        h = h + jnp.dot(m, w2lk[kk], preferred_element_type=jnp.float32)
    h2 = jnp.maximum(h, 0.0)
    h2o[...] = h2
    y3o[...] = jnp.dot(h2, w4l[...], preferred_element_type=jnp.float32)


def _tc3_body(ap, inv, h2, w4r, b4t, sel, oo):
    a = ap[...]
    iv = inv[...][:, 0:1]
    selk = sel[...]
    o = (jnp.dot(h2[...], w4r[...], preferred_element_type=jnp.float32)
         + b4t[...][0:1, :])
    for kk in range(3):
        m = (a[0, kk] + a[1, kk]) * iv
        o = o + jnp.dot(m, selk[kk], preferred_element_type=jnp.float32)
    oo[...] = o[:, :40]


def _rows(lanes):
    return pl.BlockSpec((BLK, lanes), lambda i: (i, 0))


def _chunked(nch):
    return pl.BlockSpec((NSC, nch, BLK, 16), lambda i: (0, 0, i, 0))


def _full(shape):
    return pl.BlockSpec(shape, lambda i: tuple(0 for _ in shape))


def _jnp_agg(table, src, dst, mult, nch):
    E2 = E // 2
    outs = []
    for c in range(2):
        sl = slice(c * E2, (c + 1) * E2)
        rows = []
        for k in range(nch):
            msgs = table[src[sl] * mult + k]
            rows.append(jax.ops.segment_sum(msgs, dst[sl], num_segments=NPAD))
        outs.append(jnp.stack(rows))
    return jnp.stack(outs)


def kernel(z, edge_index, W1_l, b1, W1_r, W2_l, b2, W2_r, W4_l, b4, W4_r):
    f32 = jnp.float32
    zp = jnp.concatenate(
        [z, jnp.ones((N, 1), f32), jnp.zeros((N, 12), f32)], axis=1)
    W1lp = jnp.zeros((16, 64), f32).at[0:3].set(W1_l.T)
    W1rp = jnp.zeros((16, 64), f32).at[0:3].set(W1_r.T).at[3].set(b1)
    w2l = W2_l.T.reshape(4, 16, 64)
    w2r = W2_r.T
    w4l = jnp.zeros((64, 48), f32).at[:, :40].set(W4_l.T)
    b2t = jnp.tile(b2[None, :], (8, 1))
    w4r = jnp.zeros((64, 48), f32).at[:, :40].set(W4_r.T)
    b4t = jnp.zeros((8, 48), f32).at[:, :40].set(jnp.tile(b4[None, :], (8, 1)))
    sel = jnp.stack([
        jnp.pad(jnp.eye(16, dtype=f32), ((0, 0), (16 * k, 48 - 16 * (k + 1))))
        for k in range(3)
    ])
    src = edge_index[0]
    dst = edge_index[1]
    zeros_hbm = jnp.zeros((ZROWS, 16), f32)
    padn = E_PAD - E
    src2d = jnp.concatenate(
        [src, jnp.zeros((padn,), jnp.int32)]).reshape(E_PAD // 128, 128)
    dst2d = jnp.concatenate(
        [dst, jnp.full((padn,), JUNK, jnp.int32)]).reshape(E_PAD // 128, 128)

    agg1 = _sc_agg_l1(zp, src2d, dst2d, zeros_hbm)
    h1, inv16 = pl.pallas_call(
        _tc1_body,
        grid=(GRID,),
        in_specs=[_chunked(1), _rows(16), _full((16, 64)), _full((16, 64))],
        out_specs=[_rows(64), _rows(16)],
        out_shape=[jax.ShapeDtypeStruct((N, 64), f32),
                   jax.ShapeDtypeStruct((N, 16), f32)],
    )(agg1, zp, W1lp, W1rp)

    agg2 = _sc_agg_l2(h1.reshape(4 * N, 16), src2d, dst2d, zeros_hbm)
    h2, y3 = pl.pallas_call(
        _tc2_body,
        grid=(GRID,),
        in_specs=[_chunked(4), _rows(16), _rows(64), _full((4, 16, 64)),
                  _full((64, 64)), _full((64, 48)), _full((8, 64))],
        out_specs=[_rows(64), _rows(48)],
        out_shape=[jax.ShapeDtypeStruct((N, 64), f32),
                   jax.ShapeDtypeStruct((N, 48), f32)],
    )(agg2, inv16, h1, w2l, w2r, w4l, b2t)

    agg3 = _sc_agg_l3(y3.reshape(3 * N, 16), src2d, dst2d, zeros_hbm)
    out = pl.pallas_call(
        _tc3_body,
        grid=(GRID,),
        in_specs=[_chunked(3), _rows(16), _rows(64), _full((64, 48)),
                  _full((8, 48)), _full((3, 16, 48))],
        out_specs=_rows(40),
        out_shape=jax.ShapeDtypeStruct((N, 40), f32),
    )(agg3, inv16, h2, w4r, b4t, sel)
    return out


# speedup vs baseline: 9.3512x; 1.1975x over previous
import jax
import jax.numpy as jnp
from jax import lax
from jax.experimental import pallas as pl
from jax.experimental.pallas import tpu as pltpu
from jax.experimental.pallas import tpu_sc as plsc

N = 100000
E = 1600000

NSC = 2
NTILE = 16
K = 4
B = K * 128
NB2 = 49
ET = NB2 * 2 * B
E_PAD = NSC * NTILE * ET
NPAD = 100096
ACC_ROWS = NPAD
JUNK = N
ZROWS = 1564
CPR = NPAD // NTILE


def _make_sc_agg(nch):
    def body(table, src2d, dst2d, zeros_hbm, out, idx0, idx1, dst0, dst1,
             gidx0, gidx1, rows0, rows1, acc, semg, sems0, sems1):
        c = lax.axis_index("c")
        s = lax.axis_index("s")
        w = c * NTILE + s
        tile_row0 = w * (ET // 128)

        def compute_gidx(gidx, idx, k):
            if k == 0:
                return idx
            for j in range(K):
                for i in range(8):
                    sl = pl.ds(i * 16, 16)
                    gidx[j, sl] = idx[j, sl] + (k * N)
            return gidx

        for k in range(nch):
            for zz in range(4):
                pltpu.sync_copy(
                    zeros_hbm, acc.at[pl.ds(s * CPR + zz * ZROWS, ZROWS)])
            plsc.subcore_barrier()

            def batch2(t, carry):
                row0 = tile_row0 + t * 2 * K
                @pl.when(t > 0)
                def _():
                    for j in range(K):
                        pltpu.make_async_copy(
                            rows0.at[j], acc.at[dst0.at[j]], sems0).wait()
                pltpu.sync_copy(src2d.at[pl.ds(row0, K)], idx0)
                pltpu.sync_copy(dst2d.at[pl.ds(row0, K)], dst0)
                g0 = compute_gidx(gidx0, idx0, k)
                h0 = [pltpu.async_copy(table.at[g0.at[j]], rows0.at[j], semg)
                      for j in range(K)]
                @pl.when(t > 0)
                def _():
                    for j in range(K):
                        pltpu.make_async_copy(
                            rows1.at[j], acc.at[dst1.at[j]], sems1).wait()
                pltpu.sync_copy(src2d.at[pl.ds(row0 + K, K)], idx1)
                pltpu.sync_copy(dst2d.at[pl.ds(row0 + K, K)], dst1)
                g1 = compute_gidx(gidx1, idx1, k)
                h1 = [pltpu.async_copy(table.at[g1.at[j]], rows1.at[j], semg)
                      for j in range(K)]
                for h in h0:
                    h.wait()
                for j in range(K):
                    pltpu.async_copy(rows0.at[j], acc.at[dst0.at[j]], sems0,
                                     add=True)
                for h in h1:
                    h.wait()
                for j in range(K):
                    pltpu.async_copy(rows1.at[j], acc.at[dst1.at[j]], sems1,
                                     add=True)
                return carry

            lax.fori_loop(0, NB2, batch2, 0)
            for j in range(K):
                pltpu.make_async_copy(
                    rows0.at[j], acc.at[dst0.at[j]], sems0).wait()
            for j in range(K):
                pltpu.make_async_copy(
                    rows1.at[j], acc.at[dst1.at[j]], sems1).wait()
            plsc.subcore_barrier()
            pltpu.sync_copy(acc.at[pl.ds(s * CPR, CPR)],
                            out.at[c, k, pl.ds(s * CPR, CPR)])
            plsc.subcore_barrier()

    return pl.kernel(
        body,
        out_type=jax.ShapeDtypeStruct((NSC, nch, NPAD, 16), jnp.float32),
        mesh=plsc.VectorSubcoreMesh(core_axis_name="c", subcore_axis_name="s"),
        compiler_params=pltpu.CompilerParams(use_tc_tiling_on_sc=False),
        scratch_types=[
            pltpu.VMEM((K, 128), jnp.int32),
            pltpu.VMEM((K, 128), jnp.int32),
            pltpu.VMEM((K, 128), jnp.int32),
            pltpu.VMEM((K, 128), jnp.int32),
            pltpu.VMEM((K, 128), jnp.int32),
            pltpu.VMEM((K, 128), jnp.int32),
            pltpu.VMEM((K, 128, 16), jnp.float32),
            pltpu.VMEM((K, 128, 16), jnp.float32),
            pltpu.VMEM_SHARED((ACC_ROWS, 16), jnp.float32),
            pltpu.SemaphoreType.DMA,
            pltpu.SemaphoreType.DMA,
            pltpu.SemaphoreType.DMA,
        ],
    )


_sc_agg_l1 = _make_sc_agg(1)
_sc_agg_l2 = _make_sc_agg(4)
_sc_agg_l3 = _make_sc_agg(3)

BLKR = 128
NR = N // 8
NRP = NPAD // 8
GRID = (NR + BLKR - 1) // BLKR


def _tcB_body(ap, zp, w1l, w1r, sbc, h1o, invo):
    a4 = ap[...]
    a = a4[0, 0] + a4[1, 0]
    degb = jnp.dot(a, sbc[...], preferred_element_type=jnp.float32)
    inv = 1.0 / jnp.maximum(degb, 1.0)
    m = a * inv
    zr = zp[...]
    w1lk = w1l[...]
    w1rk = w1r[...]
    for co in range(4):
        h = (jnp.dot(m, w1lk[co], preferred_element_type=jnp.float32)
             + jnp.dot(zr, w1rk[co], preferred_element_type=jnp.float32))
        h1o[co, :, :] = jnp.maximum(h, 0.0)
    invo[...] = inv


def _tcD_body(ap, inv, h1, w2l, w2r, w4l, b2bc, h2o, y3o):
    a = ap[...]
    iv = inv[...]
    h1r = h1[...]
    w2lk = w2l[...]
    w2rk = w2r[...]
    w4lk = w4l[...]
    b2k = b2bc[...]
    ms = [(a[0, ci] + a[1, ci]) * iv for ci in range(4)]
    h2s = []
    for co in range(4):
        acc = b2k[co, 0:1, :]
        for ci in range(4):
            acc = (acc
                   + jnp.dot(ms[ci], w2lk[ci, co],
                             preferred_element_type=jnp.float32)
                   + jnp.dot(h1r[ci], w2rk[ci, co],
                             preferred_element_type=jnp.float32))
        h2c = jnp.maximum(acc, 0.0)
        h2s.append(h2c)
        h2o[co, :, :] = h2c
    for co in range(3):
        y = jnp.dot(h2s[0], w4lk[0, co], preferred_element_type=jnp.float32)
        for ci in range(1, 4):
            y = y + jnp.dot(h2s[ci], w4lk[ci, co],
                            preferred_element_type=jnp.float32)
        y3o[co, :, :] = y


def _tcF_body(ap, inv, h2, w4r, b4bc, oo):
    a = ap[...]
    iv = inv[...]
    h2r = h2[...]
    w4rk = w4r[...]
    b4k = b4bc[...]
    for co in range(3):
        acc = b4k[co, 0:1, :] + (a[0, co] + a[1, co]) * iv
        for ci in range(4):
            acc = acc + jnp.dot(h2r[ci], w4rk[ci, co],
                                preferred_element_type=jnp.float32)
        oo[co, :, :] = acc


def _rows():
    return pl.BlockSpec((BLKR, 128), lambda i: (i, 0))


def _cm(nch):
    return pl.BlockSpec((nch, BLKR, 128), lambda i: (0, i, 0))


def _agg_spec(nch):
    return pl.BlockSpec((NSC, nch, BLKR, 128), lambda i: (0, 0, i, 0))


def _full(shape):
    return pl.BlockSpec(shape, lambda i: tuple(0 for _ in shape))


def kernel(z, edge_index, W1_l, b1, W1_r, W2_l, b2, W2_r, W4_l, b4, W4_r):
    f32 = jnp.float32
    eye8 = jnp.eye(8, dtype=f32)

    def bd8(w16):
        return jnp.kron(eye8, w16)

    zp = jnp.concatenate(
        [z, jnp.ones((N, 1), f32), jnp.zeros((N, 12), f32)], axis=1)
    W1lp = jnp.zeros((16, 64), f32).at[0:3].set(W1_l.T)
    W1rp = jnp.zeros((16, 64), f32).at[0:3].set(W1_r.T).at[3].set(b1)
    w1l_bd = jnp.stack([bd8(W1lp[:, 16 * co:16 * co + 16]) for co in range(4)])
    w1r_bd = jnp.stack([bd8(W1rp[:, 16 * co:16 * co + 16]) for co in range(4)])
    w2l_t = W2_l.T
    w2r_t = W2_r.T
    w2l_bd = jnp.stack([
        jnp.stack([bd8(w2l_t[16 * ci:16 * ci + 16, 16 * co:16 * co + 16])
                   for co in range(4)]) for ci in range(4)])
    w2r_bd = jnp.stack([
        jnp.stack([bd8(w2r_t[16 * ci:16 * ci + 16, 16 * co:16 * co + 16])
                   for co in range(4)]) for ci in range(4)])
    w4l48 = jnp.zeros((64, 48), f32).at[:, :40].set(W4_l.T)
    w4r48 = jnp.zeros((64, 48), f32).at[:, :40].set(W4_r.T)
    w4l_bd = jnp.stack([
        jnp.stack([bd8(w4l48[16 * ci:16 * ci + 16, 16 * co:16 * co + 16])
                   for co in range(3)]) for ci in range(4)])
    w4r_bd = jnp.stack([
        jnp.stack([bd8(w4r48[16 * ci:16 * ci + 16, 16 * co:16 * co + 16])
                   for co in range(3)]) for ci in range(4)])
    sbc = bd8(jnp.zeros((16, 16), f32).at[3].set(1.0))
    b2bc = jnp.tile(b2.reshape(4, 1, 16), (1, 8, 8))
    b4p = jnp.zeros((48,), f32).at[:40].set(b4)
    b4bc = jnp.tile(b4p.reshape(3, 1, 16), (1, 8, 8))
    src = edge_index[0]
    dst = edge_index[1]
    zeros_hbm = jnp.zeros((ZROWS, 16), f32)
    padn = E_PAD - E
    src2d = jnp.concatenate(
        [src, jnp.zeros((padn,), jnp.int32)]).reshape(E_PAD // 128, 128)
    dst2d = jnp.concatenate(
        [dst, jnp.full((padn,), JUNK, jnp.int32)]).reshape(E_PAD // 128, 128)

    agg1 = _sc_agg_l1(zp, src2d, dst2d, zeros_hbm)
    h1cm, inv16 = pl.pallas_call(
        _tcB_body,
        grid=(GRID,),
        in_specs=[_agg_spec(1), _rows(), _full((4, 128, 128)),
                  _full((4, 128, 128)), _full((128, 128))],
        out_specs=[_cm(4), _rows()],
        out_shape=[jax.ShapeDtypeStruct((4, NR, 128), f32),
                   jax.ShapeDtypeStruct((NR, 128), f32)],
    )(agg1.reshape(NSC, 1, NRP, 128), zp.reshape(NR, 128), w1l_bd, w1r_bd,
      sbc)

    agg2 = _sc_agg_l2(h1cm.reshape(4 * N, 16), src2d, dst2d, zeros_hbm)
    h2cm, y3cm = pl.pallas_call(
        _tcD_body,
        grid=(GRID,),
        in_specs=[_agg_spec(4), _rows(), _cm(4), _full((4, 4, 128, 128)),
                  _full((4, 4, 128, 128)), _full((4, 3, 128, 128)),
                  _full((4, 8, 128))],
        out_specs=[_cm(4), _cm(3)],
        out_shape=[jax.ShapeDtypeStruct((4, NR, 128), f32),
                   jax.ShapeDtypeStruct((3, NR, 128), f32)],
    )(agg2.reshape(NSC, 4, NRP, 128), inv16, h1cm, w2l_bd, w2r_bd, w4l_bd,
      b2bc)

    agg3 = _sc_agg_l3(y3cm.reshape(3 * N, 16), src2d, dst2d, zeros_hbm)
    ocm = pl.pallas_call(
        _tcF_body,
        grid=(GRID,),
        in_specs=[_agg_spec(3), _rows(), _cm(4), _full((4, 3, 128, 128)),
                  _full((3, 8, 128))],
        out_specs=_cm(3),
        out_shape=jax.ShapeDtypeStruct((3, NR, 128), f32),
    )(agg3.reshape(NSC, 3, NRP, 128), inv16, h2cm, w4r_bd, b4bc)
    o3 = ocm.reshape(3, N, 16)
    return jnp.concatenate([o3[0], o3[1], o3[2][:, :8]], axis=1)


# speedup vs baseline: 10.4049x; 1.1127x over previous
import jax
import jax.numpy as jnp
from jax import lax
from jax.experimental import pallas as pl
from jax.experimental.pallas import tpu as pltpu
from jax.experimental.pallas import tpu_sc as plsc

N = 100000
E = 1600000

NSC = 2
NTILE = 16
K = 4
B = K * 128
NB2 = 49
ET = NB2 * 2 * B
E_PAD = NSC * NTILE * ET
NPAD = 100096
ACC_ROWS = NPAD
JUNK = N
ZROWS = 1564
CPR = NPAD // NTILE


def _make_sc_agg(nch):
    def body(table, src2d, dst2d, zeros_hbm, out, idx0, idx1, dst0, dst1,
             gidx0, gidx1, rows0, rows1, acc, semg, sems0, sems1):
        c = lax.axis_index("c")
        s = lax.axis_index("s")
        w = c * NTILE + s
        tile_row0 = w * (ET // 128)

        def compute_gidx(gidx, idx, k):
            if k == 0:
                return idx
            for j in range(K):
                for i in range(8):
                    sl = pl.ds(i * 16, 16)
                    gidx[j, sl] = idx[j, sl] + (k * N)
            return gidx

        for k in range(nch):
            for zz in range(4):
                pltpu.sync_copy(
                    zeros_hbm, acc.at[pl.ds(s * CPR + zz * ZROWS, ZROWS)])
            plsc.subcore_barrier()

            def batch2(t, carry):
                row0 = tile_row0 + t * 2 * K
                @pl.when(t > 0)
                def _():
                    for j in range(K):
                        pltpu.make_async_copy(
                            rows0.at[j], acc.at[dst0.at[j]], sems0).wait()
                pltpu.sync_copy(src2d.at[pl.ds(row0, K)], idx0)
                pltpu.sync_copy(dst2d.at[pl.ds(row0, K)], dst0)
                g0 = compute_gidx(gidx0, idx0, k)
                h0 = [pltpu.async_copy(table.at[g0.at[j]], rows0.at[j], semg)
                      for j in range(K)]
                @pl.when(t > 0)
                def _():
                    for j in range(K):
                        pltpu.make_async_copy(
                            rows1.at[j], acc.at[dst1.at[j]], sems1).wait()
                pltpu.sync_copy(src2d.at[pl.ds(row0 + K, K)], idx1)
                pltpu.sync_copy(dst2d.at[pl.ds(row0 + K, K)], dst1)
                g1 = compute_gidx(gidx1, idx1, k)
                h1 = [pltpu.async_copy(table.at[g1.at[j]], rows1.at[j], semg)
                      for j in range(K)]
                for h in h0:
                    h.wait()
                for j in range(K):
                    pltpu.async_copy(rows0.at[j], acc.at[dst0.at[j]], sems0,
                                     add=True)
                for h in h1:
                    h.wait()
                for j in range(K):
                    pltpu.async_copy(rows1.at[j], acc.at[dst1.at[j]], sems1,
                                     add=True)
                return carry

            lax.fori_loop(0, NB2, batch2, 0)
            for j in range(K):
                pltpu.make_async_copy(
                    rows0.at[j], acc.at[dst0.at[j]], sems0).wait()
            for j in range(K):
                pltpu.make_async_copy(
                    rows1.at[j], acc.at[dst1.at[j]], sems1).wait()
            plsc.subcore_barrier()
            pltpu.sync_copy(acc.at[pl.ds(s * CPR, CPR)],
                            out.at[c, k, pl.ds(s * CPR, CPR)])
            plsc.subcore_barrier()

    return pl.kernel(
        body,
        out_type=jax.ShapeDtypeStruct((NSC, nch, NPAD, 16), jnp.float32),
        mesh=plsc.VectorSubcoreMesh(core_axis_name="c", subcore_axis_name="s"),
        compiler_params=pltpu.CompilerParams(use_tc_tiling_on_sc=False),
        scratch_types=[
            pltpu.VMEM((K, 128), jnp.int32),
            pltpu.VMEM((K, 128), jnp.int32),
            pltpu.VMEM((K, 128), jnp.int32),
            pltpu.VMEM((K, 128), jnp.int32),
            pltpu.VMEM((K, 128), jnp.int32),
            pltpu.VMEM((K, 128), jnp.int32),
            pltpu.VMEM((K, 128, 16), jnp.float32),
            pltpu.VMEM((K, 128, 16), jnp.float32),
            pltpu.VMEM_SHARED((ACC_ROWS, 16), jnp.float32),
            pltpu.SemaphoreType.DMA,
            pltpu.SemaphoreType.DMA,
            pltpu.SemaphoreType.DMA,
        ],
    )


_sc_agg_l1 = _make_sc_agg(1)
_sc_agg_l2 = _make_sc_agg(4)
_sc_agg_l3 = _make_sc_agg(3)

BLKR = 128
NR = N // 8
NRP = NPAD // 8
GRID = (NR + BLKR - 1) // BLKR


def _tcB_body(ap, zp, w1l, w1r, sbc, h1o, invo):
    a4 = ap[...]
    a = a4[0, 0] + a4[1, 0]
    degb = jnp.dot(a, sbc[...], preferred_element_type=jnp.float32)
    inv = 1.0 / jnp.maximum(degb, 1.0)
    m = a * inv
    zr = zp[...]
    w1lk = w1l[...]
    w1rk = w1r[...]
    for co in range(4):
        h = (jnp.dot(m, w1lk[co], preferred_element_type=jnp.float32)
             + jnp.dot(zr, w1rk[co], preferred_element_type=jnp.float32))
        h1o[co, :, :] = jnp.maximum(h, 0.0)
    invo[...] = inv


def _tcD_body(ap, inv, h1, w2l, w2r, w4l, b2bc, h2o, y3o):
    a = ap[...]
    iv = inv[...]
    h1r = h1[...]
    w2lk = w2l[...]
    w2rk = w2r[...]
    w4lk = w4l[...]
    b2k = b2bc[...]
    ms = [(a[0, ci] + a[1, ci]) * iv for ci in range(4)]
    h2s = []
    for co in range(4):
        acc = b2k[co, 0:1, :]
        for ci in range(4):
            acc = (acc
                   + jnp.dot(ms[ci], w2lk[ci, co],
                             preferred_element_type=jnp.float32)
                   + jnp.dot(h1r[ci], w2rk[ci, co],
                             preferred_element_type=jnp.float32))
        h2c = jnp.maximum(acc, 0.0)
        h2s.append(h2c)
        h2o[co, :, :] = h2c
    for co in range(3):
        y = jnp.dot(h2s[0], w4lk[0, co], preferred_element_type=jnp.float32)
        for ci in range(1, 4):
            y = y + jnp.dot(h2s[ci], w4lk[ci, co],
                            preferred_element_type=jnp.float32)
        y3o[co, :, :] = y


def _tcF_body(ap, inv, h2, w4r, b4bc, perm, oo):
    a = ap[...]
    iv = inv[...]
    h2r = h2[...]
    w4rk = w4r[...]
    b4k = b4bc[...]
    accs = []
    for co in range(3):
        acc = b4k[co, 0:1, :] + (a[0, co] + a[1, co]) * iv
        for ci in range(4):
            acc = acc + jnp.dot(h2r[ci], w4rk[ci, co],
                                preferred_element_type=jnp.float32)
        accs.append(acc)
    cat = jnp.concatenate(accs, axis=1)
    oo[...] = jnp.dot(cat, perm[...], preferred_element_type=jnp.float32)


def _rows():
    return pl.BlockSpec((BLKR, 128), lambda i: (i, 0))


def _cm(nch):
    return pl.BlockSpec((nch, BLKR, 128), lambda i: (0, i, 0))


def _agg_spec(nch):
    return pl.BlockSpec((NSC, nch, BLKR, 128), lambda i: (0, 0, i, 0))


def _full(shape):
    return pl.BlockSpec(shape, lambda i: tuple(0 for _ in shape))


def kernel(z, edge_index, W1_l, b1, W1_r, W2_l, b2, W2_r, W4_l, b4, W4_r):
    f32 = jnp.float32
    eye8 = jnp.eye(8, dtype=f32)

    def bd8(w16):
        return jnp.kron(eye8, w16)

    zp = jnp.concatenate(
        [z, jnp.ones((N, 1), f32), jnp.zeros((N, 12), f32)], axis=1)
    W1lp = jnp.zeros((16, 64), f32).at[0:3].set(W1_l.T)
    W1rp = jnp.zeros((16, 64), f32).at[0:3].set(W1_r.T).at[3].set(b1)
    w1l_bd = jnp.stack([bd8(W1lp[:, 16 * co:16 * co + 16]) for co in range(4)])
    w1r_bd = jnp.stack([bd8(W1rp[:, 16 * co:16 * co + 16]) for co in range(4)])
    w2l_t = W2_l.T
    w2r_t = W2_r.T
    w2l_bd = jnp.stack([
        jnp.stack([bd8(w2l_t[16 * ci:16 * ci + 16, 16 * co:16 * co + 16])
                   for co in range(4)]) for ci in range(4)])
    w2r_bd = jnp.stack([
        jnp.stack([bd8(w2r_t[16 * ci:16 * ci + 16, 16 * co:16 * co + 16])
                   for co in range(4)]) for ci in range(4)])
    w4l48 = jnp.zeros((64, 48), f32).at[:, :40].set(W4_l.T)
    w4r48 = jnp.zeros((64, 48), f32).at[:, :40].set(W4_r.T)
    w4l_bd = jnp.stack([
        jnp.stack([bd8(w4l48[16 * ci:16 * ci + 16, 16 * co:16 * co + 16])
                   for co in range(3)]) for ci in range(4)])
    w4r_bd = jnp.stack([
        jnp.stack([bd8(w4r48[16 * ci:16 * ci + 16, 16 * co:16 * co + 16])
                   for co in range(3)]) for ci in range(4)])
    sbc = bd8(jnp.zeros((16, 16), f32).at[3].set(1.0))
    b2bc = jnp.tile(b2.reshape(4, 1, 16), (1, 8, 8))
    b4p = jnp.zeros((48,), f32).at[:40].set(b4)
    b4bc = jnp.tile(b4p.reshape(3, 1, 16), (1, 8, 8))
    src = edge_index[0]
    dst = edge_index[1]
    zeros_hbm = jnp.zeros((ZROWS, 16), f32)
    padn = E_PAD - E
    src2d = jnp.concatenate(
        [src, jnp.zeros((padn,), jnp.int32)]).reshape(E_PAD // 128, 128)
    dst2d = jnp.concatenate(
        [dst, jnp.full((padn,), JUNK, jnp.int32)]).reshape(E_PAD // 128, 128)

    agg1 = _sc_agg_l1(zp, src2d, dst2d, zeros_hbm)
    h1cm, inv16 = pl.pallas_call(
        _tcB_body,
        grid=(GRID,),
        in_specs=[_agg_spec(1), _rows(), _full((4, 128, 128)),
                  _full((4, 128, 128)), _full((128, 128))],
        out_specs=[_cm(4), _rows()],
        out_shape=[jax.ShapeDtypeStruct((4, NR, 128), f32),
                   jax.ShapeDtypeStruct((NR, 128), f32)],
    )(agg1.reshape(NSC, 1, NRP, 128), zp.reshape(NR, 128), w1l_bd, w1r_bd,
      sbc)

    agg2 = _sc_agg_l2(h1cm.reshape(4 * N, 16), src2d, dst2d, zeros_hbm)
    h2cm, y3cm = pl.pallas_call(
        _tcD_body,
        grid=(GRID,),
        in_specs=[_agg_spec(4), _rows(), _cm(4), _full((4, 4, 128, 128)),
                  _full((4, 4, 128, 128)), _full((4, 3, 128, 128)),
                  _full((4, 8, 128))],
        out_specs=[_cm(4), _cm(3)],
        out_shape=[jax.ShapeDtypeStruct((4, NR, 128), f32),
                   jax.ShapeDtypeStruct((3, NR, 128), f32)],
    )(agg2.reshape(NSC, 4, NRP, 128), inv16, h1cm, w2l_bd, w2r_bd, w4l_bd,
      b2bc)

    agg3 = _sc_agg_l3(y3cm.reshape(3 * N, 16), src2d, dst2d, zeros_hbm)
    import numpy as _np
    rows_i, cols_i = [], []
    for k in range(3):
        for sl in range(8):
            for t in range(16):
                rows_i.append(128 * k + 16 * sl + t)
                cols_i.append(48 * sl + 16 * k + t)
    perm = jnp.zeros((384, 384), f32).at[
        _np.array(rows_i), _np.array(cols_i)].set(1.0)
    o48 = pl.pallas_call(
        _tcF_body,
        grid=(GRID,),
        in_specs=[_agg_spec(3), _rows(), _cm(4), _full((4, 3, 128, 128)),
                  _full((3, 8, 128)), _full((384, 384))],
        out_specs=pl.BlockSpec((BLKR, 384), lambda i: (i, 0)),
        out_shape=jax.ShapeDtypeStruct((NR, 384), f32),
    )(agg3.reshape(NSC, 3, NRP, 128), inv16, h2cm, w4r_bd, b4bc, perm)
    return o48.reshape(N, 48)[:, :40]
